# baseline (speedup 1.0000x reference)
import functools

import jax
import jax.numpy as jnp
from jax import lax
from jax.experimental import pallas as pl
from jax.experimental.pallas import tpu as pltpu
from jax.experimental.pallas import tpu_sc as plsc

D = 128
NW = 32
C = 112
NSUB = 14
CHUNK = NSUB * C
TOT = NW * CHUNK


def _sc_gather_sum(n, idx_hbm, m_hbm, p_hbm, out_hbm, idx_v,
                   bm0, bm1, bp0, bp1, row_v, sem0, sem1):
    wid = lax.axis_index("s") * 2 + lax.axis_index("c")
    base = pl.multiple_of(wid * CHUNK, 8)
    tail = n - (NW - 1) * CHUNK
    ntail = CHUNK - tail

    @pl.when(wid < NW - 1)
    def _():
        pltpu.sync_copy(idx_hbm.at[pl.ds(base, CHUNK)], idx_v)

    @pl.when(wid == NW - 1)
    def _():
        zeros = jnp.zeros((16,), jnp.int32)
        for k in range(ntail // 16):
            idx_v[pl.ds(tail + 16 * k, 16)] = zeros
        pltpu.sync_copy(idx_hbm.at[pl.ds(base, tail)], idx_v.at[pl.ds(0, tail)])

    bm, bp, sems = (bm0, bm1), (bp0, bp1), (sem0, sem1)

    def start(j):
        s = j % 2
        ids = idx_v.at[pl.ds(j * C, C)]
        return (pltpu.async_copy(m_hbm.at[ids], bm[s], sems[s]),
                pltpu.async_copy(p_hbm.at[ids], bp[s], sems[s]))

    accs = tuple(jnp.zeros((16,), jnp.float32) for _ in range(16))
    pending = start(0)
    for j in range(NSUB):
        nxt = start(j + 1) if j + 1 < NSUB else None
        for d in pending:
            d.wait()
        s = j % 2

        def body(i, a, _bm=bm[s], _bp=bp[s]):
            new = [a[k] + _bm[i, pl.ds(16 * k, 16)] for k in range(8)]
            new += [a[8 + k] + _bp[i, pl.ds(16 * k, 16)] for k in range(8)]
            return tuple(new)

        accs = lax.fori_loop(0, C, body, accs)
        pending = nxt
    for k in range(16):
        row_v[pl.ds(16 * k, 16)] = accs[k]
    pltpu.sync_copy(row_v, out_hbm.at[wid])


def _gather_sums(idx, m_emb, pre):
    n = idx.shape[0]
    mesh = plsc.VectorSubcoreMesh(core_axis_name="c", subcore_axis_name="s")
    f = pl.kernel(
        functools.partial(_sc_gather_sum, n),
        mesh=mesh,
        out_type=jax.ShapeDtypeStruct((NW, 2 * D), jnp.float32),
        scratch_types=[
            pltpu.VMEM((CHUNK,), jnp.int32),
            pltpu.VMEM((C, D), jnp.float32),
            pltpu.VMEM((C, D), jnp.float32),
            pltpu.VMEM((C, D), jnp.float32),
            pltpu.VMEM((C, D), jnp.float32),
            pltpu.VMEM((2 * D,), jnp.float32),
            pltpu.SemaphoreType.DMA,
            pltpu.SemaphoreType.DMA,
        ],
    )
    return f(idx, m_emb, pre)


def _combine_body(parts_ref, w_ref, bias_ref, m0_ref, p0_ref, out_ref, *, n, pad):
    s = jnp.sum(parts_ref[:, :], axis=0, keepdims=True)
    s_m = s[:, :D] - jnp.float32(pad) * m0_ref[0:1, :]
    s_p = s[:, D:] - jnp.float32(pad) * p0_ref[0:1, :]
    y = lax.dot_general(s_m, w_ref[:, :], (((1,), (1,)), ((), ())),
                        preferred_element_type=jnp.float32)
    out_ref[:, :] = s_p + y + jnp.float32(n) * bias_ref[:, :]


def kernel(medicine_it, m_embeddings, pretrained_model, W, bias):
    n = medicine_it.shape[0]
    pad = TOT - n
    parts = _gather_sums(medicine_it, m_embeddings, pretrained_model)
    out = pl.pallas_call(
        functools.partial(_combine_body, n=n, pad=pad),
        out_shape=jax.ShapeDtypeStruct((1, D), jnp.float32),
        grid=(1,),
        in_specs=[
            pl.BlockSpec((NW, 2 * D), lambda i: (0, 0)),
            pl.BlockSpec((D, D), lambda i: (0, 0)),
            pl.BlockSpec((1, D), lambda i: (0, 0)),
            pl.BlockSpec((8, D), lambda i: (0, 0)),
            pl.BlockSpec((8, D), lambda i: (0, 0)),
        ],
        out_specs=pl.BlockSpec((1, D), lambda i: (0, 0)),
    )(parts, W, bias.reshape(1, D), m_embeddings, pretrained_model)
    return out.reshape(1, 1, D)


# speedup vs baseline: 1.0416x; 1.0416x over previous
import functools

import jax
import jax.numpy as jnp
from jax import lax
from jax.experimental import pallas as pl
from jax.experimental.pallas import tpu as pltpu
from jax.experimental.pallas import tpu_sc as plsc

D = 128
NW = 32
C = 112
NSUB = 14
CHUNK = NSUB * C
TOT = NW * CHUNK


NBUF = 3


def _sc_gather_sum(n, idx_hbm, m_hbm, p_hbm, out_hbm, idx_v,
                   bm0, bm1, bm2, bp0, bp1, bp2, row_v, sem0, sem1, sem2):
    wid = lax.axis_index("s") * 2 + lax.axis_index("c")
    base = pl.multiple_of(wid * CHUNK, 8)
    tail = n - (NW - 1) * CHUNK
    ntail = CHUNK - tail

    @pl.when(wid < NW - 1)
    def _():
        pltpu.sync_copy(idx_hbm.at[pl.ds(base, CHUNK)], idx_v)

    @pl.when(wid == NW - 1)
    def _():
        zeros = jnp.zeros((16,), jnp.int32)
        for k in range(ntail // 16):
            idx_v[pl.ds(tail + 16 * k, 16)] = zeros
        pltpu.sync_copy(idx_hbm.at[pl.ds(base, tail)], idx_v.at[pl.ds(0, tail)])

    bm, bp, sems = (bm0, bm1, bm2), (bp0, bp1, bp2), (sem0, sem1, sem2)

    def start(j):
        s = j % NBUF
        ids = idx_v.at[pl.ds(j * C, C)]
        return (pltpu.async_copy(m_hbm.at[ids], bm[s], sems[s]),
                pltpu.async_copy(p_hbm.at[ids], bp[s], sems[s]))

    accs = tuple(jnp.zeros((16,), jnp.float32) for _ in range(16))
    inflight = [start(j) for j in range(NBUF - 1)]
    for j in range(NSUB):
        if j + NBUF - 1 < NSUB:
            inflight.append(start(j + NBUF - 1))
        for d in inflight.pop(0):
            d.wait()
        s = j % NBUF

        def body(i, a, _bm=bm[s], _bp=bp[s]):
            new = [a[k] + _bm[i, pl.ds(16 * k, 16)] for k in range(8)]
            new += [a[8 + k] + _bp[i, pl.ds(16 * k, 16)] for k in range(8)]
            return tuple(new)

        accs = lax.fori_loop(0, C, body, accs)
    for k in range(16):
        row_v[pl.ds(16 * k, 16)] = accs[k]
    pltpu.sync_copy(row_v, out_hbm.at[wid])


def _gather_sums(idx, m_emb, pre):
    n = idx.shape[0]
    mesh = plsc.VectorSubcoreMesh(core_axis_name="c", subcore_axis_name="s")
    f = pl.kernel(
        functools.partial(_sc_gather_sum, n),
        mesh=mesh,
        out_type=jax.ShapeDtypeStruct((NW, 2 * D), jnp.float32),
        scratch_types=[
            pltpu.VMEM((CHUNK,), jnp.int32),
            pltpu.VMEM((C, D), jnp.float32),
            pltpu.VMEM((C, D), jnp.float32),
            pltpu.VMEM((C, D), jnp.float32),
            pltpu.VMEM((C, D), jnp.float32),
            pltpu.VMEM((C, D), jnp.float32),
            pltpu.VMEM((C, D), jnp.float32),
            pltpu.VMEM((2 * D,), jnp.float32),
            pltpu.SemaphoreType.DMA,
            pltpu.SemaphoreType.DMA,
            pltpu.SemaphoreType.DMA,
        ],
    )
    return f(idx, m_emb, pre)


def _combine_body(parts_ref, w_ref, bias_ref, m0_ref, p0_ref, out_ref, *, n, pad):
    s = jnp.sum(parts_ref[:, :], axis=0, keepdims=True)
    s_m = s[:, :D] - jnp.float32(pad) * m0_ref[0:1, :]
    s_p = s[:, D:] - jnp.float32(pad) * p0_ref[0:1, :]
    y = lax.dot_general(s_m, w_ref[:, :], (((1,), (1,)), ((), ())),
                        preferred_element_type=jnp.float32)
    out_ref[:, :] = s_p + y + jnp.float32(n) * bias_ref[:, :]


def kernel(medicine_it, m_embeddings, pretrained_model, W, bias):
    n = medicine_it.shape[0]
    pad = TOT - n
    parts = _gather_sums(medicine_it, m_embeddings, pretrained_model)
    out = pl.pallas_call(
        functools.partial(_combine_body, n=n, pad=pad),
        out_shape=jax.ShapeDtypeStruct((1, D), jnp.float32),
        grid=(1,),
        in_specs=[
            pl.BlockSpec((NW, 2 * D), lambda i: (0, 0)),
            pl.BlockSpec((D, D), lambda i: (0, 0)),
            pl.BlockSpec((1, D), lambda i: (0, 0)),
            pl.BlockSpec((8, D), lambda i: (0, 0)),
            pl.BlockSpec((8, D), lambda i: (0, 0)),
        ],
        out_specs=pl.BlockSpec((1, D), lambda i: (0, 0)),
    )(parts, W, bias.reshape(1, D), m_embeddings, pretrained_model)
    return out.reshape(1, 1, D)
